# baseline (speedup 1.0000x reference)
import functools

import jax
import jax.numpy as jnp
from jax.experimental import pallas as pl
from jax.experimental.pallas import tpu as pltpu

_THRESH = 0.1
_BM = 2048
_BN = 256
_L = 128


def _prep_kernel(apad_ref, xrow_ref, yrow_ref, an_ref, bn_ref, dh_ref):
    a = apad_ref[...]
    ones = jnp.ones((_L, _L), jnp.float32)
    n2 = jax.lax.dot_general(a * a, ones, (((1,), (0,)), ((), ())),
                             preferred_element_type=jnp.float32)
    inv = 1.0 / jnp.maximum(jnp.sqrt(n2), 1e-6)
    an = a * inv
    an_ref[...] = an.astype(jnp.bfloat16)

    q = n2 * inv * inv
    hf = jnp.where(q > _THRESH, 1.0, 0.0)
    dh_ref[...] = jnp.sum(hf.reshape(_BM // 8, 8, _L), axis=0).reshape(1, 8, _L)

    rx = xrow_ref[...]
    ry = yrow_ref[...]
    rinv = 1.0 / jnp.maximum(jnp.sqrt(rx * rx + ry * ry), 1e-6)
    bn = jnp.concatenate(
        [rx * rinv, ry * rinv, jnp.zeros((_L - 2, _BM), jnp.float32)], axis=0)
    bn_ref[...] = bn.astype(jnp.bfloat16)


def _chunk(a_ref, bn_ref, idx):
    b = bn_ref[:, pl.ds(idx, _BN)]
    t32 = jax.lax.dot_general(a_ref[...], b, (((1,), (0,)), ((), ())),
                              preferred_element_type=jnp.float32)
    t = t32.astype(jnp.bfloat16)
    hf = jnp.where(jnp.abs(t) > jnp.bfloat16(_THRESH),
                   jnp.bfloat16(1.0), jnp.bfloat16(0.0))
    m = _BM
    while m > 16:
        m //= 2
        hf = hf[:m] + hf[m:]
    return hf.astype(jnp.float32)


@jax.jit
def kernel(node_pos, edge_index):
    e = edge_index.shape[1]
    d = node_pos[edge_index[1]] - node_pos[edge_index[0]]
    apad = jnp.pad(d, ((0, 0), (0, _L - 2)))
    xrow = d[:, 0][None, :]
    yrow = d[:, 1][None, :]
    g = e // _BM

    an, bn, dh = pl.pallas_call(
        _prep_kernel,
        grid=(g,),
        in_specs=[
            pl.BlockSpec((_BM, _L), lambda i: (i, 0)),
            pl.BlockSpec((1, _BM), lambda i: (0, i)),
            pl.BlockSpec((1, _BM), lambda i: (0, i)),
        ],
        out_specs=[
            pl.BlockSpec((_BM, _L), lambda i: (i, 0)),
            pl.BlockSpec((_L, _BM), lambda i: (0, i)),
            pl.BlockSpec((1, 8, _L), lambda i: (i, 0, 0)),
        ],
        out_shape=[
            jax.ShapeDtypeStruct((e, _L), jnp.bfloat16),
            jax.ShapeDtypeStruct((_L, e), jnp.bfloat16),
            jax.ShapeDtypeStruct((g, 8, _L), jnp.float32),
        ],
        compiler_params=pltpu.CompilerParams(
            dimension_semantics=("arbitrary",)),
    )(apad, xrow, yrow)

    nchunks = e // _BN
    mesh = pltpu.create_tensorcore_mesh("core")
    ncores = mesh.devices.shape[0]
    nper = g // ncores

    def run(refs):
        an_hbm, bn_hbm, out_hbm = refs

        @pl.core_map(
            mesh,
            scratch_shapes=[
                pltpu.VMEM((_L, e), jnp.bfloat16),
                pltpu.VMEM((_BM, _L), jnp.bfloat16),
                pltpu.VMEM((1, 16, _BN), jnp.float32),
                pltpu.SemaphoreType.DMA,
                pltpu.SemaphoreType.DMA,
                pltpu.SemaphoreType.DMA,
            ])
        def _(bn_vmem, a_vmem, out_vmem, sem_b, sem_a, sem_o):
            core = jax.lax.axis_index("core")
            cp = pltpu.make_async_copy(bn_hbm, bn_vmem, sem_b)
            cp.start()
            cp.wait()
            for ib in range(nper):
                i = core * nper + ib
                cpa = pltpu.make_async_copy(
                    an_hbm.at[pl.ds(i * _BM, _BM), :], a_vmem, sem_a)
                cpa.start()
                cpa.wait()
                out_vmem[...] = jnp.zeros_like(out_vmem)

                def body(c, carry):
                    base = pl.multiple_of(c * 2 * _BN, 2 * _BN)
                    r0 = _chunk(a_vmem, bn_vmem, base)
                    r1 = _chunk(a_vmem, bn_vmem, base + _BN)
                    out_vmem[...] += (r0 + r1).reshape(1, 16, _BN)
                    return carry

                jax.lax.fori_loop(0, nchunks // 2, body, 0)
                cpo = pltpu.make_async_copy(
                    out_vmem, out_hbm.at[pl.ds(i, 1), :, :], sem_o)
                cpo.start()
                cpo.wait()

    out_init = jnp.zeros((g, 16, _BN), jnp.float32)
    _, _, out = pl.run_state(run)((an, bn, out_init))

    total = jnp.sum(out)
    diag = jnp.sum(dh) / _L
    denom = e * (e - 1) / 2
    return (total - diag) * 0.5 / denom


# speedup vs baseline: 1.1406x; 1.1406x over previous
import functools

import jax
import jax.numpy as jnp
from jax.experimental import pallas as pl
from jax.experimental.pallas import tpu as pltpu

_THRESH = 0.1
_BM = 2048
_BN = 256
_L = 128


def _prep_kernel(apad_ref, xrow_ref, yrow_ref, an_ref, bn_ref, dh_ref):
    a = apad_ref[...]
    ones = jnp.ones((_L, _L), jnp.float32)
    n2 = jax.lax.dot_general(a * a, ones, (((1,), (0,)), ((), ())),
                             preferred_element_type=jnp.float32)
    inv = 1.0 / jnp.maximum(jnp.sqrt(n2), 1e-6)
    an = a * inv
    an_ref[...] = an.astype(jnp.bfloat16)

    q = n2 * inv * inv
    hf = jnp.where(q > _THRESH, 1.0, 0.0)
    dh_ref[...] = jnp.sum(hf.reshape(_BM // 8, 8, _L), axis=0).reshape(1, 8, _L)

    rx = xrow_ref[...]
    ry = yrow_ref[...]
    rinv = 1.0 / jnp.maximum(jnp.sqrt(rx * rx + ry * ry), 1e-6)
    bn = jnp.concatenate(
        [rx * rinv, ry * rinv, jnp.zeros((_L - 2, _BM), jnp.float32)], axis=0)
    bn_ref[...] = bn.astype(jnp.bfloat16)


def _chunk(a_ref, bn_ref, idx):
    b = bn_ref[:, pl.ds(idx, _BN)]
    t32 = jax.lax.dot_general(a_ref[...], b, (((1,), (0,)), ((), ())),
                              preferred_element_type=jnp.float32)
    t = t32.astype(jnp.bfloat16)
    hf = jnp.where(jnp.abs(t) > jnp.bfloat16(_THRESH),
                   jnp.bfloat16(1.0), jnp.bfloat16(0.0))
    m = _BM
    while m > 16:
        m //= 2
        hf = hf[:m] + hf[m:]
    return hf.astype(jnp.float32)


_UNROLL = 8


def _count_kernel(nchunks, an_ref, bn_ref, out_ref, acc_ref):
    acc_ref[...] = jnp.zeros_like(acc_ref)

    def body(c, carry):
        base = pl.multiple_of(c * _UNROLL * _BN, _UNROLL * _BN)
        total = _chunk(an_ref, bn_ref, base)
        for u in range(1, _UNROLL):
            total = total + _chunk(an_ref, bn_ref, base + u * _BN)
        acc_ref[...] += total
        return carry

    jax.lax.fori_loop(0, nchunks // _UNROLL, body, 0)
    out_ref[...] = acc_ref[...].reshape(1, 16, _BN)


@jax.jit
def kernel(node_pos, edge_index):
    e = edge_index.shape[1]
    d = node_pos[edge_index[1]] - node_pos[edge_index[0]]
    apad = jnp.pad(d, ((0, 0), (0, _L - 2)))
    xrow = d[:, 0][None, :]
    yrow = d[:, 1][None, :]
    g = e // _BM

    an, bn, dh = pl.pallas_call(
        _prep_kernel,
        grid=(g,),
        in_specs=[
            pl.BlockSpec((_BM, _L), lambda i: (i, 0)),
            pl.BlockSpec((1, _BM), lambda i: (0, i)),
            pl.BlockSpec((1, _BM), lambda i: (0, i)),
        ],
        out_specs=[
            pl.BlockSpec((_BM, _L), lambda i: (i, 0)),
            pl.BlockSpec((_L, _BM), lambda i: (0, i)),
            pl.BlockSpec((1, 8, _L), lambda i: (i, 0, 0)),
        ],
        out_shape=[
            jax.ShapeDtypeStruct((e, _L), jnp.bfloat16),
            jax.ShapeDtypeStruct((_L, e), jnp.bfloat16),
            jax.ShapeDtypeStruct((g, 8, _L), jnp.float32),
        ],
        compiler_params=pltpu.CompilerParams(
            dimension_semantics=("arbitrary",)),
    )(apad, xrow, yrow)

    out = pl.pallas_call(
        functools.partial(_count_kernel, e // _BN),
        grid=(g,),
        in_specs=[
            pl.BlockSpec((_BM, _L), lambda i: (i, 0)),
            pl.BlockSpec((_L, e), lambda i: (0, 0)),
        ],
        out_specs=pl.BlockSpec((1, 16, _BN), lambda i: (i, 0, 0)),
        out_shape=jax.ShapeDtypeStruct((g, 16, _BN), jnp.float32),
        scratch_shapes=[pltpu.VMEM((16, _BN), jnp.float32)],
        compiler_params=pltpu.CompilerParams(
            dimension_semantics=("arbitrary",)),
    )(an, bn)

    total = jnp.sum(out)
    diag = jnp.sum(dh) / _L
    denom = e * (e - 1) / 2
    return (total - diag) * 0.5 / denom
